# Initial kernel scaffold; baseline (speedup 1.0000x reference)
import jax
import jax.numpy as jnp
from jax.experimental import pallas as pl


def kernel(vfeats, text_features, concept_ids, v_w, v_b):
    raise NotImplementedError("write your pallas kernel here")


# speedup vs baseline: 1.6035x; 1.6035x over previous
import functools

import jax
import jax.numpy as jnp
from jax.experimental import pallas as pl
from jax.experimental.pallas import tpu as pltpu

_NEG_INF = float("-inf")
_BIG_I32 = 2**30
_TOPK = 10


def _body(vf_ref, txt_ref, cid_ref, vwt_ref, vb_ref,
          out_v_ref, out_i_ref, out_a_ref,
          img_ref, rv_ref, ri_ref, ra_ref, m_ref, s_ref,
          *, num_blocks, k_blk):
    step = pl.program_id(0)
    B = vf_ref.shape[0]

    @pl.when(step == 0)
    def _init():
        img = jnp.dot(vf_ref[...], vwt_ref[...],
                      preferred_element_type=jnp.float32) + vb_ref[...]
        nrm = jnp.sqrt(jnp.sum(img * img, axis=1, keepdims=True))
        img_ref[...] = img / nrm
        rv_ref[...] = jnp.full(rv_ref.shape, _NEG_INF, jnp.float32)
        ri_ref[...] = jnp.full(ri_ref.shape, _BIG_I32, jnp.int32)
        ra_ref[...] = jnp.zeros(ra_ref.shape, jnp.int32)
        m_ref[...] = jnp.full(m_ref.shape, _NEG_INF, jnp.float32)
        s_ref[...] = jnp.zeros(s_ref.shape, jnp.float32)

    t = txt_ref[...]
    nrm = jnp.sqrt(jnp.sum(t * t, axis=1, keepdims=True))
    tn = t / nrm
    logits = 100.0 * jax.lax.dot_general(
        img_ref[...], tn, (((1,), (1,)), ((), ())),
        preferred_element_type=jnp.float32)

    bm = jnp.max(logits, axis=1, keepdims=True)
    m_new = jnp.maximum(m_ref[...], bm)
    bs = jnp.sum(jnp.exp(logits - m_new), axis=1, keepdims=True)
    s_ref[...] = s_ref[...] * jnp.exp(m_ref[...] - m_new) + bs
    m_ref[...] = m_new

    col = jax.lax.broadcasted_iota(jnp.int32, (B, k_blk), 1) + step * k_blk
    cid = jnp.broadcast_to(cid_ref[...][0], (B, k_blk))
    wv = jnp.concatenate([rv_ref[...], logits], axis=1)
    wi = jnp.concatenate([ri_ref[...], col], axis=1)
    wa = jnp.concatenate([ra_ref[...], cid], axis=1)
    for tpos in range(_TOPK):
        m = jnp.max(wv, axis=1, keepdims=True)
        ci = jnp.min(jnp.where(wv == m, wi, _BIG_I32), axis=1, keepdims=True)
        eqi = wi == ci
        av = jnp.sum(jnp.where(eqi, wa, 0), axis=1, keepdims=True)
        rv_ref[:, tpos:tpos + 1] = m
        ri_ref[:, tpos:tpos + 1] = ci
        ra_ref[:, tpos:tpos + 1] = av
        wv = jnp.where(eqi, _NEG_INF, wv)

    @pl.when(step == num_blocks - 1)
    def _finalize():
        out_v_ref[...] = jnp.exp(rv_ref[:, :_TOPK] - m_ref[...]) / s_ref[...]
        out_i_ref[...] = ri_ref[:, :_TOPK]
        out_a_ref[...] = ra_ref[:, :_TOPK]


@jax.jit
def kernel(vfeats, text_features, concept_ids, v_w, v_b):
    B, D = vfeats.shape
    K = text_features.shape[0]
    k_blk = 2000 if K % 2000 == 0 else K
    num_blocks = K // k_blk

    cid3 = concept_ids.reshape(num_blocks, 1, k_blk)
    vwt = v_w.T
    vb2 = v_b.reshape(1, D)

    body = functools.partial(_body, num_blocks=num_blocks, k_blk=k_blk)
    values, indices, attr_ids = pl.pallas_call(
        body,
        grid=(num_blocks,),
        in_specs=[
            pl.BlockSpec((B, D), lambda i: (0, 0)),
            pl.BlockSpec((k_blk, D), lambda i: (i, 0)),
            pl.BlockSpec((1, 1, k_blk), lambda i: (i, 0, 0)),
            pl.BlockSpec((D, D), lambda i: (0, 0)),
            pl.BlockSpec((1, D), lambda i: (0, 0)),
        ],
        out_specs=[
            pl.BlockSpec((B, _TOPK), lambda i: (0, 0)),
            pl.BlockSpec((B, _TOPK), lambda i: (0, 0)),
            pl.BlockSpec((B, _TOPK), lambda i: (0, 0)),
        ],
        out_shape=[
            jax.ShapeDtypeStruct((B, _TOPK), jnp.float32),
            jax.ShapeDtypeStruct((B, _TOPK), jnp.int32),
            jax.ShapeDtypeStruct((B, _TOPK), jnp.int32),
        ],
        scratch_shapes=[
            pltpu.VMEM((B, D), jnp.float32),
            pltpu.VMEM((B, 16), jnp.float32),
            pltpu.VMEM((B, 16), jnp.int32),
            pltpu.VMEM((B, 16), jnp.int32),
            pltpu.VMEM((B, 1), jnp.float32),
            pltpu.VMEM((B, 1), jnp.float32),
        ],
        compiler_params=pltpu.CompilerParams(
            dimension_semantics=("arbitrary",),
        ),
    )(vfeats, text_features, cid3, vwt, vb2)
    return values, indices, attr_ids
